# baseline (speedup 1.0000x reference)
import functools

import jax
import jax.numpy as jnp
from jax import lax
from jax.experimental import pallas as pl
from jax.experimental.pallas import tpu as pltpu
from jax.experimental.pallas import tpu_sc as plsc

N = 4096
M = 4096

_NC = 2
_NS = 16
_NW = _NC * _NS
_L = 16

_COUNTS = (8192, 4096, 2048)
_PER_TILE = tuple(c // _NW for c in _COUNTS)
_GCHUNK = tuple(min(c, 128) for c in _PER_TILE)
_NGATH = tuple(pt // gc for pt, gc in zip(_PER_TILE, _GCHUNK))


def _sc_gather_body(gt_hbm, ref0, src0, ref1, src1, ref2, src2, out_hbm,
                    idx_vs, val_vs, ridx_vs, sidx_vs, accs_v, sem):
    wid = lax.axis_index("s") * _NC + lax.axis_index("c")
    refs = (ref0, ref1, ref2)
    srcs = (src0, src1, src2)
    for a in range(3):
        cnt = _PER_TILE[a]
        gchunk = _GCHUNK[a]
        base = wid * cnt
        pltpu.sync_copy(refs[a].at[pl.ds(base, cnt)], ridx_vs[a])
        pltpu.sync_copy(srcs[a].at[pl.ds(base, cnt)], sidx_vs[a])
        for i in range(cnt // _L):
            r = ridx_vs[a][pl.ds(i * _L, _L)]
            s = sidx_vs[a][pl.ds(i * _L, _L)]
            j, l = divmod(i * _L, gchunk)
            idx_vs[a][j, pl.ds(l, _L)] = r * M + s
        cps = []
        for j in range(_NGATH[a]):
            cps.append(pltpu.async_copy(gt_hbm.at[idx_vs[a].at[j]],
                                        val_vs[a].at[j], sem))
        for cp in cps:
            cp.wait()
        acc = jnp.zeros((_L,), jnp.float32)
        for i in range(cnt // _L):
            j, l = divmod(i * _L, gchunk)
            v = val_vs[a][j, pl.ds(l, _L)]
            acc = acc + (v + 1.0) * 0.5
        accs_v[a] = acc
    pltpu.sync_copy(accs_v, out_hbm.at[wid])


def _sc_gather_sums(gt_flat, ref0, src0, ref1, src1, ref2, src2):
    mesh = plsc.VectorSubcoreMesh(core_axis_name="c", subcore_axis_name="s")
    scratch = (
        [pltpu.VMEM((_NGATH[a], _GCHUNK[a]), jnp.int32) for a in range(3)],
        [pltpu.VMEM((_NGATH[a], _GCHUNK[a]), jnp.float32) for a in range(3)],
        [pltpu.VMEM((_PER_TILE[a],), jnp.int32) for a in range(3)],
        [pltpu.VMEM((_PER_TILE[a],), jnp.int32) for a in range(3)],
        pltpu.VMEM((3, _L), jnp.float32),
        pltpu.SemaphoreType.DMA,
    )
    fn = pl.kernel(
        _sc_gather_body,
        out_type=jax.ShapeDtypeStruct((_NW, 3, _L), jnp.float32),
        mesh=mesh,
        scratch_types=scratch,
    )
    return fn(gt_flat, ref0, src0, ref1, src1, ref2, src2)


_TC_BLOCK = 256


def _tc_masked_body(gt_ref, init_ref, s_ref, c_ref):
    i = pl.program_id(0)

    @pl.when(i == 0)
    def _init():
        s_ref[0, 0] = 0.0
        c_ref[0, 0] = 0.0

    gt = (gt_ref[...] + 1.0) * 0.5
    init = (init_ref[...] + 1.0) * 0.5
    mask = init == 1.0
    s_ref[0, 0] += jnp.sum(jnp.where(mask, gt, 0.0))
    c_ref[0, 0] += jnp.sum(mask.astype(jnp.float32))


def _tc_masked_sums(gt, init):
    grid = (N // _TC_BLOCK,)
    return pl.pallas_call(
        _tc_masked_body,
        grid=grid,
        in_specs=[
            pl.BlockSpec((_TC_BLOCK, M), lambda i: (i, 0)),
            pl.BlockSpec((_TC_BLOCK, M), lambda i: (i, 0)),
        ],
        out_specs=[
            pl.BlockSpec(memory_space=pltpu.SMEM),
            pl.BlockSpec(memory_space=pltpu.SMEM),
        ],
        out_shape=[
            jax.ShapeDtypeStruct((1, 1), jnp.float32),
            jax.ShapeDtypeStruct((1, 1), jnp.float32),
        ],
    )(gt, init)


@jax.jit
def kernel(gt_corr_matrix, pred_corr, pred_corr_1_2, pred_corr_1_4,
           init_corr_matrix):
    gt_flat = gt_corr_matrix.reshape(-1)
    pairs = (pred_corr, pred_corr_1_2, pred_corr_1_4)
    refs = [p[:, 0] for p in pairs]
    srcs = [p[:, 1] for p in pairs]

    partials = _sc_gather_sums(gt_flat, refs[0], srcs[0], refs[1], srcs[1],
                               refs[2], srcs[2])
    sums = jnp.sum(partials, axis=(0, 2))
    precision = sums[0] / _COUNTS[0]
    precision_1_2 = sums[1] / _COUNTS[1]
    precision_1_4 = sums[2] / _COUNTS[2]

    s, c = _tc_masked_sums(gt_corr_matrix, init_corr_matrix)
    init_precision = s[0, 0] / jnp.maximum(c[0, 0], 1.0)

    return (precision, precision_1_2, precision_1_4, init_precision)


# speedup vs baseline: 1.6338x; 1.6338x over previous
import functools

import jax
import jax.numpy as jnp
from jax import lax
from jax.experimental import pallas as pl
from jax.experimental.pallas import tpu as pltpu
from jax.experimental.pallas import tpu_sc as plsc

N = 4096
M = 4096

_NC = 2
_NS = 16
_NW = _NC * _NS
_L = 16

_COUNTS = (8192, 4096, 2048)
_PER_TILE = tuple(c // _NW for c in _COUNTS)
_GCHUNK = tuple(min(c, 128) for c in _PER_TILE)
_NGATH = tuple(pt // gc for pt, gc in zip(_PER_TILE, _GCHUNK))


def _sc_gather_body(gt_hbm, ref0, src0, ref1, src1, ref2, src2, out_hbm,
                    idx_vs, val_vs, ridx_vs, sidx_vs, accs_v, sem):
    wid = lax.axis_index("s") * _NC + lax.axis_index("c")
    refs = (ref0, ref1, ref2)
    srcs = (src0, src1, src2)
    for a in range(3):
        cnt = _PER_TILE[a]
        gchunk = _GCHUNK[a]
        base = wid * cnt
        pltpu.sync_copy(refs[a].at[pl.ds(base, cnt)], ridx_vs[a])
        pltpu.sync_copy(srcs[a].at[pl.ds(base, cnt)], sidx_vs[a])
        for i in range(cnt // _L):
            r = ridx_vs[a][pl.ds(i * _L, _L)]
            s = sidx_vs[a][pl.ds(i * _L, _L)]
            j, l = divmod(i * _L, gchunk)
            idx_vs[a][j, pl.ds(l, _L)] = (
                ((r >> 3) * (M // 128) + (s >> 7)) * 1024
                + (r & 7) * 128 + (s & 127))
        cps = []
        for j in range(_NGATH[a]):
            cps.append(pltpu.async_copy(gt_hbm.at[idx_vs[a].at[j]],
                                        val_vs[a].at[j], sem))
        for cp in cps:
            cp.wait()
        acc = jnp.zeros((_L,), jnp.float32)
        for i in range(cnt // _L):
            j, l = divmod(i * _L, gchunk)
            v = val_vs[a][j, pl.ds(l, _L)]
            acc = acc + (v + 1.0) * 0.5
        accs_v[a] = acc
    pltpu.sync_copy(accs_v, out_hbm.at[wid])


def _sc_gather_sums(gt_flat, ref0, src0, ref1, src1, ref2, src2):
    mesh = plsc.VectorSubcoreMesh(core_axis_name="c", subcore_axis_name="s")
    scratch = (
        [pltpu.VMEM((_NGATH[a], _GCHUNK[a]), jnp.int32) for a in range(3)],
        [pltpu.VMEM((_NGATH[a], _GCHUNK[a]), jnp.float32) for a in range(3)],
        [pltpu.VMEM((_PER_TILE[a],), jnp.int32) for a in range(3)],
        [pltpu.VMEM((_PER_TILE[a],), jnp.int32) for a in range(3)],
        pltpu.VMEM((3, _L), jnp.float32),
        pltpu.SemaphoreType.DMA,
    )
    fn = pl.kernel(
        _sc_gather_body,
        out_type=jax.ShapeDtypeStruct((_NW, 3, _L), jnp.float32),
        mesh=mesh,
        scratch_types=scratch,
    )
    return fn(gt_flat, ref0, src0, ref1, src1, ref2, src2)


_TC_BLOCK = 256


def _tc_masked_body(gt_ref, init_ref, s_ref, c_ref):
    i = pl.program_id(0)

    @pl.when(i == 0)
    def _init():
        s_ref[0, 0] = 0.0
        c_ref[0, 0] = 0.0

    gt = (gt_ref[...] + 1.0) * 0.5
    init = (init_ref[...] + 1.0) * 0.5
    mask = init == 1.0
    s_ref[0, 0] += jnp.sum(jnp.where(mask, gt, 0.0))
    c_ref[0, 0] += jnp.sum(mask.astype(jnp.float32))


def _tc_masked_sums(gt, init):
    grid = (N // _TC_BLOCK,)
    return pl.pallas_call(
        _tc_masked_body,
        grid=grid,
        in_specs=[
            pl.BlockSpec((_TC_BLOCK, M), lambda i: (i, 0)),
            pl.BlockSpec((_TC_BLOCK, M), lambda i: (i, 0)),
        ],
        out_specs=[
            pl.BlockSpec(memory_space=pltpu.SMEM),
            pl.BlockSpec(memory_space=pltpu.SMEM),
        ],
        out_shape=[
            jax.ShapeDtypeStruct((1, 1), jnp.float32),
            jax.ShapeDtypeStruct((1, 1), jnp.float32),
        ],
    )(gt, init)


@jax.jit
def kernel(gt_corr_matrix, pred_corr, pred_corr_1_2, pred_corr_1_4,
           init_corr_matrix):
    gt_flat = (gt_corr_matrix.reshape(N // 8, 8, M // 128, 128)
               .transpose(0, 2, 1, 3).reshape(-1))
    pairs = (pred_corr, pred_corr_1_2, pred_corr_1_4)
    refs = [p[:, 0] for p in pairs]
    srcs = [p[:, 1] for p in pairs]

    partials = _sc_gather_sums(gt_flat, refs[0], srcs[0], refs[1], srcs[1],
                               refs[2], srcs[2])
    sums = jnp.sum(partials, axis=(0, 2))
    precision = sums[0] / _COUNTS[0]
    precision_1_2 = sums[1] / _COUNTS[1]
    precision_1_4 = sums[2] / _COUNTS[2]

    s, c = _tc_masked_sums(gt_corr_matrix, init_corr_matrix)
    init_precision = s[0, 0] / jnp.maximum(c[0, 0], 1.0)

    return (precision, precision_1_2, precision_1_4, init_precision)
